# Initial kernel scaffold; baseline (speedup 1.0000x reference)
import jax
import jax.numpy as jnp
from jax.experimental import pallas as pl


def kernel(h, edge_index, yes_no, W_t1, b_t1, W_g0, b_g0, W_g1, b_g1, W_t2, b_t2, yes_weight, no_weight):
    raise NotImplementedError("write your pallas kernel here")


# speedup vs baseline: 5.9244x; 5.9244x over previous
import functools

import jax
import jax.numpy as jnp
from jax import lax
from jax.experimental import pallas as pl
from jax.experimental.pallas import tpu as pltpu
from jax.experimental.pallas import tpu_sc as plsc

EPS = 0.3
CB = 128
NS = 16
NCU = 1
NW = NCU * NS


def _dvec(deg):
    return jnp.where(deg > 0.0, lax.rsqrt(jnp.maximum(deg, 1.0)), 0.0)


def _zsum(z_ref):
    z = z_ref[0]
    for i in range(1, z_ref.shape[0]):
        z = z + z_ref[i]
    return z


def _tc1_body(h_ref, w_ref, b_ref, wg_ref, bg_ref, deg_ref,
              h1_ref, hd_ref, ab_ref):
    y = lax.dot_general(h_ref[...], w_ref[...], (((1,), (1,)), ((), ())),
                        preferred_element_type=jnp.float32)
    y = jnp.maximum(y + b_ref[...], 0.0)
    h1_ref[...] = y
    hd_ref[...] = y * _dvec(deg_ref[...])
    ab = lax.dot_general(wg_ref[...], y, (((1,), (1,)), ((), ())),
                         preferred_element_type=jnp.float32)
    ab_ref[...] = (ab + bg_ref[...])[:, None, :]


def _tc_mid_body(h1_ref, z_ref, deg_ref, wg_ref, bg_ref, hd_ref, ab_ref):
    d = _dvec(deg_ref[...])
    h2 = EPS * h1_ref[...] + d * _zsum(z_ref)
    hd_ref[...] = d * h2
    ab = lax.dot_general(wg_ref[...], h2, (((1,), (1,)), ((), ())),
                         preferred_element_type=jnp.float32)
    ab_ref[...] = (ab + bg_ref[...])[:, None, :]


def _tc3_body(h1_ref, z_ref, deg_ref, w_ref, b_ref, out_ref):
    h3 = EPS * h1_ref[...] + _dvec(deg_ref[...]) * _zsum(z_ref)
    o = lax.dot_general(h3, w_ref[...], (((1,), (1,)), ((), ())),
                        preferred_element_type=jnp.float32)
    o = o + b_ref[...]
    m = jnp.max(o, axis=1, keepdims=True)
    s = jnp.sum(jnp.exp(o - m), axis=1, keepdims=True)
    out_ref[...] = o - m - jnp.log(s)


def _tc1(h, W_t1, b_t1, wg, bg, deg, N, HID):
    return pl.pallas_call(
        _tc1_body,
        out_shape=[jax.ShapeDtypeStruct((N, HID), jnp.float32),
                   jax.ShapeDtypeStruct((N, HID), jnp.float32),
                   jax.ShapeDtypeStruct((2, 1, N), jnp.float32)],
    )(h, W_t1, b_t1, wg, bg, deg)


def _tc_mid(h1, z, deg, wg, bg, N, HID):
    return pl.pallas_call(
        _tc_mid_body,
        out_shape=[jax.ShapeDtypeStruct((N, HID), jnp.float32),
                   jax.ShapeDtypeStruct((2, 1, N), jnp.float32)],
    )(h1, z, deg, wg, bg)


def _tc3(h1, z, deg, W_t2, b_t2, N, OUT):
    return pl.pallas_call(
        _tc3_body,
        out_shape=jax.ShapeDtypeStruct((N, OUT), jnp.float32),
    )(h1, z, deg, W_t2, b_t2)


_SC_PARAMS = pltpu.CompilerParams(needs_layout_passes=False)


@functools.lru_cache(maxsize=None)
def _make_sc_deg(N, EPWP):
    DB = 4096
    NB = EPWP // DB
    NCHB = DB // CB
    assert EPWP % DB == 0
    mesh = plsc.VectorSubcoreMesh(core_axis_name="c", subcore_axis_name="s",
                                  num_cores=NCU)

    @functools.partial(
        pl.kernel, mesh=mesh, compiler_params=_SC_PARAMS,
        out_type=jax.ShapeDtypeStruct((N,), jnp.float32),
        scratch_types=[
            pltpu.VMEM((DB,), jnp.int32),
            pltpu.VMEM((N,), jnp.float32),
            pltpu.VMEM((1, CB), jnp.int32),
            pltpu.VMEM_SHARED((N,), jnp.float32),
        ],
    )
    def sc_deg(dst_hbm, deg_out, dst_b, stage, idx2d, deg_sh):
        s = lax.axis_index("s")
        zero16 = jnp.zeros((16,), jnp.float32)
        one16 = jnp.full((16,), 1.0, jnp.float32)

        def _z(i, _):
            stage[pl.ds(i * 16, 16)] = zero16
            return 0
        lax.fori_loop(0, N // 16, _z, 0)

        @pl.when(s == 0)
        def _():
            pltpu.sync_copy(stage, deg_sh)

        def _o(i, _):
            stage[pl.ds(i * 16, 16)] = one16
            return 0
        lax.fori_loop(0, CB // 16, _o, 0)
        plsc.subcore_barrier()

        for b in range(NB):
            pltpu.sync_copy(dst_hbm.at[s, 0, pl.ds(b * DB, DB)], dst_b)

            def _dg(k, _):
                for j in range(CB // 16):
                    idx2d[0, pl.ds(j * 16, 16)] = dst_b[pl.ds(k * CB + j * 16, 16)]
                pltpu.sync_copy(stage.at[pl.ds(0, CB)],
                                deg_sh.at[idx2d.at[0]], add=True)
                return 0
            lax.fori_loop(0, NCHB, _dg, 0)
        plsc.subcore_barrier()

        @pl.when(s == 0)
        def _():
            pltpu.sync_copy(deg_sh, deg_out)

    return sc_deg


@functools.lru_cache(maxsize=None)
def _make_sc_layer(N, EPWP, D):
    BE = 1024
    NB = EPWP // BE
    NCHB = BE // CB
    ZR = ((N // NS) // CB + 1) * CB
    ZL = N - (NS - 1) * ZR
    assert EPWP % BE == 0 and 0 < ZL <= ZR and ZL % 8 == 0
    mesh = plsc.VectorSubcoreMesh(core_axis_name="c", subcore_axis_name="s",
                                  num_cores=NCU)

    @functools.partial(
        pl.kernel, mesh=mesh, compiler_params=_SC_PARAMS,
        out_type=jax.ShapeDtypeStruct((NCU, N, D), jnp.float32),
        scratch_types=[
            pltpu.VMEM((N,), jnp.float32),
            pltpu.VMEM((N,), jnp.float32),
            pltpu.VMEM((BE,), jnp.int32),
            pltpu.VMEM((BE,), jnp.int32),
            pltpu.VMEM((BE,), jnp.int32),
            pltpu.VMEM((CB,), jnp.float32),
            pltpu.VMEM((CB, D), jnp.float32),
            pltpu.VMEM((1, CB), jnp.int32),
            pltpu.VMEM((16,), jnp.float32),
            pltpu.VMEM_SHARED((N, D), jnp.float32),
        ],
    )
    def sc_layer(hd_hbm, ab_hbm, src_hbm, dst_hbm, yn_hbm, par_hbm, z_out,
                 a_loc, b_loc, src_b, dst_b, yn_b, e_chk, rows, idx2d,
                 par_v, z_sh):
        c = lax.axis_index("c")
        s = lax.axis_index("s")
        w = s * NCU + c
        zero16 = jnp.zeros((16,), jnp.float32)

        def _zr(i, _):
            rows[i // (D // 16), pl.ds((i % (D // 16)) * 16, 16)] = zero16
            return 0
        lax.fori_loop(0, CB * (D // 16), _zr, 0)

        zbase = s * ZR

        @pl.when(s < NS - 1)
        def _():
            for q in range(ZR // CB):
                pltpu.sync_copy(rows, z_sh.at[pl.ds(zbase + q * CB, CB)])

        @pl.when(s == NS - 1)
        def _():
            for q in range(ZL // CB):
                pltpu.sync_copy(rows, z_sh.at[pl.ds(zbase + q * CB, CB)])
            if ZL % CB:
                pltpu.sync_copy(
                    rows.at[pl.ds(0, ZL % CB)],
                    z_sh.at[pl.ds(zbase + (ZL // CB) * CB, ZL % CB)])

        pltpu.sync_copy(par_hbm, par_v)
        pltpu.sync_copy(ab_hbm.at[0, 0], a_loc)
        pltpu.sync_copy(ab_hbm.at[1, 0], b_loc)
        pvec = par_v[pl.ds(0, 16)]
        t_yes = pvec[0]
        t_no = pvec[1]
        plsc.subcore_barrier()

        def _blk(b, _):
            off = b * BE
            pltpu.sync_copy(src_hbm.at[w, 0, pl.ds(off, BE)], src_b)
            pltpu.sync_copy(dst_hbm.at[w, 0, pl.ds(off, BE)], dst_b)
            pltpu.sync_copy(yn_hbm.at[w, 0, pl.ds(off, BE)], yn_b)

            def _chunk(k, _):
                for j in range(CB // 16):
                    sv = src_b[pl.ds(k * CB + j * 16, 16)]
                    dv = dst_b[pl.ds(k * CB + j * 16, 16)]
                    yv = yn_b[pl.ds(k * CB + j * 16, 16)]
                    av = plsc.load_gather(a_loc, [dv])
                    bv = plsc.load_gather(b_loc, [sv])
                    x = av + bv
                    t = jnp.exp(-2.0 * jnp.abs(x))
                    m = (1.0 - t) / (1.0 + t)
                    g = jnp.where(x < 0.0, -m, m)
                    yn = jnp.where(yv == 1, t_yes, t_no)
                    e = jnp.where(yv < 2, (g + yn) * 0.5, 0.0)
                    e_chk[pl.ds(j * 16, 16)] = e

                pltpu.sync_copy(hd_hbm.at[src_b.at[pl.ds(k * CB, CB)]], rows)

                def _scale(i, _):
                    evec = e_chk[pl.ds(i * 16, 16)]
                    for rr in range(16):
                        ev = evec[rr]
                        for jj in range(D // 16):
                            sl = pl.ds(jj * 16, 16)
                            rows[i * 16 + rr, sl] = rows[i * 16 + rr, sl] * ev
                    return 0
                lax.fori_loop(0, CB // 16, _scale, 0)

                for j in range(CB // 16):
                    idx2d[0, pl.ds(j * 16, 16)] = dst_b[pl.ds(k * CB + j * 16, 16)]
                pltpu.sync_copy(rows, z_sh.at[idx2d.at[0]], add=True)
                return 0
            lax.fori_loop(0, NCHB, _chunk, 0)
            return 0
        lax.fori_loop(0, NB, _blk, 0)

        plsc.subcore_barrier()

        @pl.when(s < NS - 1)
        def _():
            pltpu.sync_copy(z_sh.at[pl.ds(zbase, ZR)],
                            z_out.at[c, pl.ds(zbase, ZR)])

        @pl.when(s == NS - 1)
        def _():
            pltpu.sync_copy(z_sh.at[pl.ds(zbase, ZL)],
                            z_out.at[c, pl.ds(zbase, ZL)])

    return sc_layer


def kernel(h, edge_index, yes_no, W_t1, b_t1, W_g0, b_g0, W_g1, b_g1,
           W_t2, b_t2, yes_weight, no_weight):
    N, _ = h.shape
    HID = W_t1.shape[0]
    OUT = W_t2.shape[0]
    E = edge_index.shape[1]
    EPW = E // NW
    EPWP = -(-EPW // 4096) * 4096
    PADW = EPWP - EPW

    def _pad(x, val):
        x2 = x.reshape(NW, EPW)
        if PADW:
            x2 = jnp.pad(x2, ((0, 0), (0, PADW)), constant_values=val)
        return x2.reshape(NW, 1, EPWP)

    src3d = _pad(edge_index[0], 0)
    dst3d = _pad(edge_index[1], 0)
    yn3d = _pad(yes_no, 2)
    wg0 = W_g0.reshape(2, HID)
    wg1 = W_g1.reshape(2, HID)
    zero1 = jnp.zeros((1,), jnp.float32)
    bg0 = jnp.concatenate([b_g0, zero1]).reshape(2, 1)
    bg1 = jnp.concatenate([b_g1, zero1]).reshape(2, 1)
    params = jnp.zeros((16,), jnp.float32)
    params = params.at[0].set(jnp.tanh(yes_weight))
    params = params.at[1].set(jnp.tanh(no_weight))

    sc_deg = _make_sc_deg(N, EPWP)
    sc_layer = _make_sc_layer(N, EPWP, HID)

    deg = sc_deg(dst3d)
    if PADW:
        deg = deg - jnp.zeros((N,), jnp.float32).at[0].set(float(NW * PADW))
    deg = deg.reshape(N, 1)
    h1, hd0, ab0 = _tc1(h, W_t1, b_t1.reshape(1, HID), wg0, bg0, deg, N, HID)
    z0 = sc_layer(hd0, ab0, src3d, dst3d, yn3d, params)
    hd1, ab1 = _tc_mid(h1, z0, deg, wg1, bg1, N, HID)
    z1 = sc_layer(hd1, ab1, src3d, dst3d, yn3d, params)
    return _tc3(h1, z1, deg, W_t2, b_t2.reshape(1, OUT), N, OUT)


# speedup vs baseline: 8.1534x; 1.3762x over previous
import functools

import jax
import jax.numpy as jnp
from jax import lax
from jax.experimental import pallas as pl
from jax.experimental.pallas import tpu as pltpu
from jax.experimental.pallas import tpu_sc as plsc

EPS = 0.3
CB = 128
NS = 16
NCU = 2
NW = NCU * NS


def _dvec(deg):
    return jnp.where(deg > 0.0, lax.rsqrt(jnp.maximum(deg, 1.0)), 0.0)


def _zsum(z_ref):
    z = z_ref[0]
    for i in range(1, z_ref.shape[0]):
        z = z + z_ref[i]
    return z


def _tc1_body(h_ref, w_ref, b_ref, wg_ref, bg_ref, deg_ref,
              h1_ref, hd_ref, ab_ref):
    y = lax.dot_general(h_ref[...], w_ref[...], (((1,), (1,)), ((), ())),
                        preferred_element_type=jnp.float32)
    y = jnp.maximum(y + b_ref[...], 0.0)
    h1_ref[...] = y
    hd_ref[...] = y * _dvec(deg_ref[...])
    ab = lax.dot_general(wg_ref[...], y, (((1,), (1,)), ((), ())),
                         preferred_element_type=jnp.float32)
    ab_ref[...] = (ab + bg_ref[...])[:, None, :]


def _tc_mid_body(h1_ref, z_ref, deg_ref, wg_ref, bg_ref, hd_ref, ab_ref):
    d = _dvec(deg_ref[...])
    h2 = EPS * h1_ref[...] + d * _zsum(z_ref)
    hd_ref[...] = d * h2
    ab = lax.dot_general(wg_ref[...], h2, (((1,), (1,)), ((), ())),
                         preferred_element_type=jnp.float32)
    ab_ref[...] = (ab + bg_ref[...])[:, None, :]


def _tc3_body(h1_ref, z_ref, deg_ref, w_ref, b_ref, out_ref):
    h3 = EPS * h1_ref[...] + _dvec(deg_ref[...]) * _zsum(z_ref)
    o = lax.dot_general(h3, w_ref[...], (((1,), (1,)), ((), ())),
                        preferred_element_type=jnp.float32)
    o = o + b_ref[...]
    m = jnp.max(o, axis=1, keepdims=True)
    s = jnp.sum(jnp.exp(o - m), axis=1, keepdims=True)
    out_ref[...] = o - m - jnp.log(s)


def _tc1(h, W_t1, b_t1, wg, bg, deg, N, HID):
    return pl.pallas_call(
        _tc1_body,
        out_shape=[jax.ShapeDtypeStruct((N, HID), jnp.float32),
                   jax.ShapeDtypeStruct((N, HID), jnp.float32),
                   jax.ShapeDtypeStruct((2, 1, N), jnp.float32)],
    )(h, W_t1, b_t1, wg, bg, deg)


def _tc_mid(h1, z, deg, wg, bg, N, HID):
    return pl.pallas_call(
        _tc_mid_body,
        out_shape=[jax.ShapeDtypeStruct((N, HID), jnp.float32),
                   jax.ShapeDtypeStruct((2, 1, N), jnp.float32)],
    )(h1, z, deg, wg, bg)


def _tc3(h1, z, deg, W_t2, b_t2, N, OUT):
    return pl.pallas_call(
        _tc3_body,
        out_shape=jax.ShapeDtypeStruct((N, OUT), jnp.float32),
    )(h1, z, deg, W_t2, b_t2)


_SC_PARAMS = pltpu.CompilerParams(needs_layout_passes=False)


@functools.lru_cache(maxsize=None)
def _make_sc_deg(N, EPWP):
    DB = 1024
    NB = EPWP // DB
    NCHB = DB // CB
    DBT = NW // NS
    assert EPWP % DB == 0
    mesh = plsc.VectorSubcoreMesh(core_axis_name="c", subcore_axis_name="s",
                                  num_cores=NCU)

    @functools.partial(
        pl.kernel, mesh=mesh, compiler_params=_SC_PARAMS,
        out_type=jax.ShapeDtypeStruct((N,), jnp.float32),
        scratch_types=[
            pltpu.VMEM((DB,), jnp.int32),
            pltpu.VMEM((N,), jnp.float32),
            pltpu.VMEM((1, CB), jnp.int32),
            pltpu.VMEM_SHARED((N,), jnp.float32),
        ],
    )
    def sc_deg(dst_hbm, deg_out, dst_b, stage, idx2d, deg_sh):
        c = lax.axis_index("c")
        s = lax.axis_index("s")
        zero16 = jnp.zeros((16,), jnp.float32)
        one16 = jnp.full((16,), 1.0, jnp.float32)

        def _z(i, _):
            stage[pl.ds(i * 16, 16)] = zero16
            return 0
        lax.fori_loop(0, N // 16, _z, 0)

        @pl.when(s == 0)
        def _():
            pltpu.sync_copy(stage, deg_sh)

        def _o(i, _):
            stage[pl.ds(i * 16, 16)] = one16
            return 0
        lax.fori_loop(0, CB // 16, _o, 0)
        plsc.subcore_barrier()

        for t in range(DBT):
            for b in range(NB):
                pltpu.sync_copy(
                    dst_hbm.at[s * DBT + t, 0, pl.ds(b * DB, DB)], dst_b)

                def _dg(k, _):
                    for j in range(CB // 16):
                        idx2d[0, pl.ds(j * 16, 16)] = \
                            dst_b[pl.ds(k * CB + j * 16, 16)]
                    pltpu.sync_copy(stage.at[pl.ds(0, CB)],
                                    deg_sh.at[idx2d.at[0]], add=True)
                    return 0
                lax.fori_loop(0, NCHB, _dg, 0)
        plsc.subcore_barrier()

        @pl.when(jnp.logical_and(s == 0, c == 0))
        def _():
            pltpu.sync_copy(deg_sh, deg_out)

    return sc_deg


@functools.lru_cache(maxsize=None)
def _make_sc_layer(N, EPWP, D):
    BE = 1024
    NB = EPWP // BE
    NCHB = BE // CB
    ZR = ((N // NS) // CB + 1) * CB
    ZL = N - (NS - 1) * ZR
    assert EPWP % BE == 0 and 0 < ZL <= ZR and ZL % 8 == 0
    mesh = plsc.VectorSubcoreMesh(core_axis_name="c", subcore_axis_name="s",
                                  num_cores=NCU)

    @functools.partial(
        pl.kernel, mesh=mesh, compiler_params=_SC_PARAMS,
        out_type=jax.ShapeDtypeStruct((NCU, N, D), jnp.float32),
        scratch_types=[
            pltpu.VMEM((N,), jnp.float32),
            pltpu.VMEM((N,), jnp.float32),
            pltpu.VMEM((BE,), jnp.int32),
            pltpu.VMEM((BE,), jnp.int32),
            pltpu.VMEM((BE,), jnp.int32),
            pltpu.VMEM((CB,), jnp.float32),
            pltpu.VMEM((CB, D), jnp.float32),
            pltpu.VMEM((1, CB), jnp.int32),
            pltpu.VMEM((16,), jnp.float32),
            pltpu.VMEM_SHARED((N, D), jnp.float32),
        ],
    )
    def sc_layer(hd_hbm, ab_hbm, src_hbm, dst_hbm, yn_hbm, par_hbm, z_out,
                 a_loc, b_loc, src_b, dst_b, yn_b, e_chk, rows, idx2d,
                 par_v, z_sh):
        c = lax.axis_index("c")
        s = lax.axis_index("s")
        w = s * NCU + c
        zero16 = jnp.zeros((16,), jnp.float32)

        def _zr(i, _):
            rows[i // (D // 16), pl.ds((i % (D // 16)) * 16, 16)] = zero16
            return 0
        lax.fori_loop(0, CB * (D // 16), _zr, 0)

        zbase = s * ZR

        @pl.when(s < NS - 1)
        def _():
            for q in range(ZR // CB):
                pltpu.sync_copy(rows, z_sh.at[pl.ds(zbase + q * CB, CB)])

        @pl.when(s == NS - 1)
        def _():
            for q in range(ZL // CB):
                pltpu.sync_copy(rows, z_sh.at[pl.ds(zbase + q * CB, CB)])
            if ZL % CB:
                pltpu.sync_copy(
                    rows.at[pl.ds(0, ZL % CB)],
                    z_sh.at[pl.ds(zbase + (ZL // CB) * CB, ZL % CB)])

        pltpu.sync_copy(par_hbm, par_v)
        pltpu.sync_copy(ab_hbm.at[0, 0], a_loc)
        pltpu.sync_copy(ab_hbm.at[1, 0], b_loc)
        pvec = par_v[pl.ds(0, 16)]
        t_yes = pvec[0]
        t_no = pvec[1]
        plsc.subcore_barrier()

        def _blk(b, _):
            off = b * BE
            pltpu.sync_copy(src_hbm.at[w, 0, pl.ds(off, BE)], src_b)
            pltpu.sync_copy(dst_hbm.at[w, 0, pl.ds(off, BE)], dst_b)
            pltpu.sync_copy(yn_hbm.at[w, 0, pl.ds(off, BE)], yn_b)

            def _chunk(k, _):
                for j in range(CB // 16):
                    sv = src_b[pl.ds(k * CB + j * 16, 16)]
                    dv = dst_b[pl.ds(k * CB + j * 16, 16)]
                    yv = yn_b[pl.ds(k * CB + j * 16, 16)]
                    av = plsc.load_gather(a_loc, [dv])
                    bv = plsc.load_gather(b_loc, [sv])
                    x = av + bv
                    t = jnp.exp(-2.0 * jnp.abs(x))
                    m = (1.0 - t) / (1.0 + t)
                    g = jnp.where(x < 0.0, -m, m)
                    yn = jnp.where(yv == 1, t_yes, t_no)
                    e = jnp.where(yv < 2, (g + yn) * 0.5, 0.0)
                    e_chk[pl.ds(j * 16, 16)] = e

                pltpu.sync_copy(hd_hbm.at[src_b.at[pl.ds(k * CB, CB)]], rows)

                def _scale(i, _):
                    evec = e_chk[pl.ds(i * 16, 16)]
                    for rr in range(16):
                        ev = evec[rr]
                        for jj in range(D // 16):
                            sl = pl.ds(jj * 16, 16)
                            rows[i * 16 + rr, sl] = rows[i * 16 + rr, sl] * ev
                    return 0
                lax.fori_loop(0, CB // 16, _scale, 0)

                for j in range(CB // 16):
                    idx2d[0, pl.ds(j * 16, 16)] = dst_b[pl.ds(k * CB + j * 16, 16)]
                pltpu.sync_copy(rows, z_sh.at[idx2d.at[0]], add=True)
                return 0
            lax.fori_loop(0, NCHB, _chunk, 0)
            return 0
        lax.fori_loop(0, NB, _blk, 0)

        plsc.subcore_barrier()

        @pl.when(s < NS - 1)
        def _():
            pltpu.sync_copy(z_sh.at[pl.ds(zbase, ZR)],
                            z_out.at[c, pl.ds(zbase, ZR)])

        @pl.when(s == NS - 1)
        def _():
            pltpu.sync_copy(z_sh.at[pl.ds(zbase, ZL)],
                            z_out.at[c, pl.ds(zbase, ZL)])

    return sc_layer


def kernel(h, edge_index, yes_no, W_t1, b_t1, W_g0, b_g0, W_g1, b_g1,
           W_t2, b_t2, yes_weight, no_weight):
    N, _ = h.shape
    HID = W_t1.shape[0]
    OUT = W_t2.shape[0]
    E = edge_index.shape[1]
    EPW = E // NW
    EPWP = -(-EPW // 1024) * 1024
    PADW = EPWP - EPW

    def _pad(x, val):
        x2 = x.reshape(NW, EPW)
        if PADW:
            x2 = jnp.pad(x2, ((0, 0), (0, PADW)), constant_values=val)
        return x2.reshape(NW, 1, EPWP)

    src3d = _pad(edge_index[0], 0)
    dst3d = _pad(edge_index[1], 0)
    yn3d = _pad(yes_no, 2)
    wg0 = W_g0.reshape(2, HID)
    wg1 = W_g1.reshape(2, HID)
    zero1 = jnp.zeros((1,), jnp.float32)
    bg0 = jnp.concatenate([b_g0, zero1]).reshape(2, 1)
    bg1 = jnp.concatenate([b_g1, zero1]).reshape(2, 1)
    params = jnp.zeros((16,), jnp.float32)
    params = params.at[0].set(jnp.tanh(yes_weight))
    params = params.at[1].set(jnp.tanh(no_weight))

    sc_deg = _make_sc_deg(N, EPWP)
    sc_layer = _make_sc_layer(N, EPWP, HID)

    deg = sc_deg(dst3d)
    if PADW:
        deg = deg - jnp.zeros((N,), jnp.float32).at[0].set(float(NW * PADW))
    deg = deg.reshape(N, 1)
    h1, hd0, ab0 = _tc1(h, W_t1, b_t1.reshape(1, HID), wg0, bg0, deg, N, HID)
    z0 = sc_layer(hd0, ab0, src3d, dst3d, yn3d, params)
    hd1, ab1 = _tc_mid(h1, z0, deg, wg1, bg1, N, HID)
    z1 = sc_layer(hd1, ab1, src3d, dst3d, yn3d, params)
    return _tc3(h1, z1, deg, W_t2, b_t2.reshape(1, OUT), N, OUT)
